# Initial kernel scaffold; baseline (speedup 1.0000x reference)
import functools

import jax
import jax.numpy as jnp
from jax import lax
from jax.experimental import pallas as pl
from jax.experimental.pallas import tpu as pltpu
from jax.experimental.pallas import tpu_sc as plsc

N = 10000
E = 320000
IN_FEATS = 128
D = 64
ALPHA = 0.1
K = 10

NC = 2
TPC = 16
NW = NC * TPC
CW = 128
CPW = 80
EPW = CPW * CW
EP = NW * EPW
NP = 10240
RPT = NP // TPC
ROW_BLK = 128


def _sc_degree_body(dst_hbm, d0_hbm, d1_hbm, deg_sp, dstbuf, ones_v, zrow):
    c = lax.axis_index("c")
    s = lax.axis_index("s")
    w = s * NC + c

    @pl.loop(0, RPT, step=16)
    def _(i):
        zrow[pl.ds(i, 16)] = jnp.zeros((16,), jnp.float32)

    @pl.loop(0, CW, step=16)
    def _(i):
        ones_v[pl.ds(i, 16)] = jnp.ones((16,), jnp.float32)

    pltpu.sync_copy(zrow, deg_sp.at[pl.ds(s * RPT, RPT)])
    plsc.subcore_barrier()

    pltpu.sync_copy(dst_hbm.at[w], dstbuf)

    @pl.loop(0, CPW)
    def _(j):
        pltpu.sync_copy(ones_v, deg_sp.at[dstbuf.at[j]], add=True)

    plsc.subcore_barrier()

    pltpu.sync_copy(deg_sp.at[pl.ds(s * RPT, RPT)], zrow)

    @pl.when(c == 0)
    def _():
        pltpu.sync_copy(zrow, d0_hbm.at[pl.ds(s * RPT, RPT)])

    @pl.when(c == 1)
    def _():
        pltpu.sync_copy(zrow, d1_hbm.at[pl.ds(s * RPT, RPT)])


@jax.jit
def _sc_degree(dst3):
    mesh = plsc.VectorSubcoreMesh(core_axis_name="c", subcore_axis_name="s")
    f = pl.kernel(
        _sc_degree_body,
        out_type=(
            jax.ShapeDtypeStruct((NP,), jnp.float32),
            jax.ShapeDtypeStruct((NP,), jnp.float32),
        ),
        mesh=mesh,
        scratch_types=[
            pltpu.VMEM_SHARED((NP,), jnp.float32),
            pltpu.VMEM((CPW, CW), jnp.int32),
            pltpu.VMEM((CW,), jnp.float32),
            pltpu.VMEM((RPT,), jnp.float32),
        ],
    )
    return f(dst3)


def _sc_step_body(g_hbm, src_hbm, dst_hbm, p0_hbm, p1_hbm,
                  agg_sp, srcbuf, dstbuf, rows0, rows1, zblk, sem0, sem1):
    c = lax.axis_index("c")
    s = lax.axis_index("s")
    w = s * NC + c

    @pl.loop(0, ROW_BLK)
    def _(r):
        @pl.loop(0, D, step=16)
        def _(cc):
            zblk[r, pl.ds(cc, 16)] = jnp.zeros((16,), jnp.float32)

    @pl.loop(0, RPT, step=ROW_BLK)
    def _(b):
        pltpu.sync_copy(zblk, agg_sp.at[pl.ds(s * RPT + b, ROW_BLK)])

    pltpu.sync_copy(src_hbm.at[w], srcbuf)
    pltpu.sync_copy(dst_hbm.at[w], dstbuf)
    plsc.subcore_barrier()

    pltpu.async_copy(g_hbm.at[srcbuf.at[0]], rows0, sem0)

    @pl.loop(0, CPW, step=2)
    def _(j):
        pltpu.async_copy(g_hbm.at[srcbuf.at[j + 1]], rows1, sem1)
        pltpu.make_async_copy(g_hbm.at[pl.ds(0, CW)], rows0, sem0).wait()
        pltpu.sync_copy(rows0, agg_sp.at[dstbuf.at[j]], add=True)

        @pl.when(j + 2 < CPW)
        def _():
            pltpu.async_copy(g_hbm.at[srcbuf.at[j + 2]], rows0, sem0)

        pltpu.make_async_copy(g_hbm.at[pl.ds(0, CW)], rows1, sem1).wait()
        pltpu.sync_copy(rows1, agg_sp.at[dstbuf.at[j + 1]], add=True)

    plsc.subcore_barrier()

    @pl.loop(0, RPT, step=ROW_BLK)
    def _(b):
        pltpu.sync_copy(agg_sp.at[pl.ds(s * RPT + b, ROW_BLK)], zblk)

        @pl.when(c == 0)
        def _():
            pltpu.sync_copy(zblk, p0_hbm.at[pl.ds(s * RPT + b, ROW_BLK)])

        @pl.when(c == 1)
        def _():
            pltpu.sync_copy(zblk, p1_hbm.at[pl.ds(s * RPT + b, ROW_BLK)])


@jax.jit
def _sc_step(g, src3, dst3):
    mesh = plsc.VectorSubcoreMesh(core_axis_name="c", subcore_axis_name="s")
    f = pl.kernel(
        _sc_step_body,
        out_type=(
            jax.ShapeDtypeStruct((NP, D), jnp.float32),
            jax.ShapeDtypeStruct((NP, D), jnp.float32),
        ),
        mesh=mesh,
        scratch_types=[
            pltpu.VMEM_SHARED((NP, D), jnp.float32),
            pltpu.VMEM((CPW, CW), jnp.int32),
            pltpu.VMEM((CPW, CW), jnp.int32),
            pltpu.VMEM((CW, D), jnp.float32),
            pltpu.VMEM((CW, D), jnp.float32),
            pltpu.VMEM((ROW_BLK, D), jnp.float32),
            pltpu.SemaphoreType.DMA,
            pltpu.SemaphoreType.DMA,
        ],
    )
    return f(g, src3, dst3)


_MLP_BLK = 512


def _tc_mlp_prep_body(x_ref, w1_ref, b1_ref, w2_ref, b2_ref, d0_ref, d1_ref,
                      g0_ref, a1_ref, bb1_ref, a2_ref, bb2_ref):
    h = jnp.dot(x_ref[...], w1_ref[...], preferred_element_type=jnp.float32)
    h = jnp.maximum(h + b1_ref[...], 0.0)
    h = jnp.dot(h, w2_ref[...], preferred_element_type=jnp.float32) + b2_ref[...]
    deg = d0_ref[...] + d1_ref[...]
    norm = lax.rsqrt(jnp.maximum(deg, 1.0))
    g0_ref[...] = h * norm
    a1_ref[...] = (1.0 - ALPHA) * norm * norm
    bb1_ref[...] = (ALPHA * norm) * h
    a2_ref[...] = (1.0 - ALPHA) * norm
    bb2_ref[...] = ALPHA * h


@jax.jit
def _tc_mlp_prep(xp, W1, b1, W2, b2, d0, d1):
    grid = (NP // _MLP_BLK,)
    return pl.pallas_call(
        _tc_mlp_prep_body,
        grid=grid,
        in_specs=[
            pl.BlockSpec((_MLP_BLK, IN_FEATS), lambda i: (i, 0)),
            pl.BlockSpec((IN_FEATS, IN_FEATS), lambda i: (0, 0)),
            pl.BlockSpec((1, IN_FEATS), lambda i: (0, 0)),
            pl.BlockSpec((IN_FEATS, D), lambda i: (0, 0)),
            pl.BlockSpec((1, D), lambda i: (0, 0)),
            pl.BlockSpec((_MLP_BLK, 1), lambda i: (i, 0)),
            pl.BlockSpec((_MLP_BLK, 1), lambda i: (i, 0)),
        ],
        out_specs=(
            pl.BlockSpec((_MLP_BLK, D), lambda i: (i, 0)),
            pl.BlockSpec((_MLP_BLK, 1), lambda i: (i, 0)),
            pl.BlockSpec((_MLP_BLK, D), lambda i: (i, 0)),
            pl.BlockSpec((_MLP_BLK, 1), lambda i: (i, 0)),
            pl.BlockSpec((_MLP_BLK, D), lambda i: (i, 0)),
        ),
        out_shape=(
            jax.ShapeDtypeStruct((NP, D), jnp.float32),
            jax.ShapeDtypeStruct((NP, 1), jnp.float32),
            jax.ShapeDtypeStruct((NP, D), jnp.float32),
            jax.ShapeDtypeStruct((NP, 1), jnp.float32),
            jax.ShapeDtypeStruct((NP, D), jnp.float32),
        ),
    )(xp, W1, b1.reshape(1, IN_FEATS), W2, b2.reshape(1, D), d0, d1)


def _tc_combine_body(p0_ref, p1_ref, a_ref, b_ref, o_ref):
    o_ref[...] = a_ref[...] * (p0_ref[...] + p1_ref[...]) + b_ref[...]


@jax.jit
def _tc_combine(p0, p1, a, b):
    grid = (NP // _MLP_BLK,)
    return pl.pallas_call(
        _tc_combine_body,
        grid=grid,
        in_specs=[
            pl.BlockSpec((_MLP_BLK, D), lambda i: (i, 0)),
            pl.BlockSpec((_MLP_BLK, D), lambda i: (i, 0)),
            pl.BlockSpec((_MLP_BLK, 1), lambda i: (i, 0)),
            pl.BlockSpec((_MLP_BLK, D), lambda i: (i, 0)),
        ],
        out_specs=pl.BlockSpec((_MLP_BLK, D), lambda i: (i, 0)),
        out_shape=jax.ShapeDtypeStruct((NP, D), jnp.float32),
    )(p0, p1, a, b)


def kernel(features, W1, b1, W2, b2, edge_index):
    xp = jnp.pad(features, ((0, NP - N), (0, 0)))
    src = edge_index[0]
    dst = edge_index[1]
    pad_idx = N + (jnp.arange(EP - E, dtype=jnp.int32) % (NP - N))
    src3 = jnp.concatenate([src, pad_idx]).reshape(NW, CPW, CW)
    dst3 = jnp.concatenate([dst, pad_idx]).reshape(NW, CPW, CW)

    d0, d1 = _sc_degree(dst3)
    g, A1, B1, A2, B2 = _tc_mlp_prep(
        xp, W1, b1, W2, b2, d0.reshape(NP, 1), d1.reshape(NP, 1))

    for t in range(K):
        p0, p1 = _sc_step(g, src3, dst3)
        if t < K - 1:
            g = _tc_combine(p0, p1, A1, B1)
        else:
            g = _tc_combine(p0, p1, A2, B2)
    return g[:N]


# speedup vs baseline: 12.7837x; 12.7837x over previous
import functools

import jax
import jax.numpy as jnp
from jax import lax
from jax.experimental import pallas as pl
from jax.experimental.pallas import tpu as pltpu
from jax.experimental.pallas import tpu_sc as plsc

N = 10000
E = 320000
IN_FEATS = 128
D = 64
ALPHA = 0.1
K = 10

NC = 2
TPC = 16
NW = NC * TPC
CW = 128
CPW = 80
EPW = CPW * CW
EP = NW * EPW
NP = 10240
RPT = NP // TPC
ROW_BLK = 128


def _sc_degree_body(dst_hbm, d0_hbm, d1_hbm, deg_sp, dstbuf, ones_v, zrow):
    c = lax.axis_index("c")
    s = lax.axis_index("s")
    w = s * NC + c

    @pl.loop(0, RPT, step=16)
    def _(i):
        zrow[pl.ds(i, 16)] = jnp.zeros((16,), jnp.float32)

    @pl.loop(0, CW, step=16)
    def _(i):
        ones_v[pl.ds(i, 16)] = jnp.ones((16,), jnp.float32)

    pltpu.sync_copy(zrow, deg_sp.at[pl.ds(s * RPT, RPT)])
    plsc.subcore_barrier()

    pltpu.sync_copy(dst_hbm.at[w], dstbuf)

    @pl.loop(0, CPW)
    def _(j):
        pltpu.sync_copy(ones_v, deg_sp.at[dstbuf.at[j]], add=True)

    plsc.subcore_barrier()

    pltpu.sync_copy(deg_sp.at[pl.ds(s * RPT, RPT)], zrow)

    @pl.when(c == 0)
    def _():
        pltpu.sync_copy(zrow, d0_hbm.at[pl.ds(s * RPT, RPT)])

    @pl.when(c == 1)
    def _():
        pltpu.sync_copy(zrow, d1_hbm.at[pl.ds(s * RPT, RPT)])


@jax.jit
def _sc_degree(dst3):
    mesh = plsc.VectorSubcoreMesh(core_axis_name="c", subcore_axis_name="s")
    f = pl.kernel(
        _sc_degree_body,
        out_type=(
            jax.ShapeDtypeStruct((NP,), jnp.float32),
            jax.ShapeDtypeStruct((NP,), jnp.float32),
        ),
        mesh=mesh,
        scratch_types=[
            pltpu.VMEM_SHARED((NP,), jnp.float32),
            pltpu.VMEM((CPW, CW), jnp.int32),
            pltpu.VMEM((CW,), jnp.float32),
            pltpu.VMEM((RPT,), jnp.float32),
        ],
    )
    return f(dst3)


def _sc_step_body(g_hbm, src_hbm, dst_hbm, p0_hbm, p1_hbm,
                  agg_sp, srcbuf, dstbuf, rows0, rows1, zblk, sem0, sem1):
    c = lax.axis_index("c")
    s = lax.axis_index("s")
    w = s * NC + c

    @pl.loop(0, ROW_BLK)
    def _(r):
        @pl.loop(0, D, step=16)
        def _(cc):
            zblk[r, pl.ds(cc, 16)] = jnp.zeros((16,), jnp.float32)

    @pl.loop(0, RPT, step=ROW_BLK)
    def _(b):
        pltpu.sync_copy(zblk, agg_sp.at[pl.ds(s * RPT + b, ROW_BLK)])

    pltpu.sync_copy(src_hbm.at[w], srcbuf)
    pltpu.sync_copy(dst_hbm.at[w], dstbuf)
    plsc.subcore_barrier()

    pltpu.async_copy(g_hbm.at[srcbuf.at[0]], rows0, sem0)

    @pl.loop(0, CPW, step=2)
    def _(j):
        pltpu.async_copy(g_hbm.at[srcbuf.at[j + 1]], rows1, sem1)
        pltpu.make_async_copy(g_hbm.at[pl.ds(0, CW)], rows0, sem0).wait()
        pltpu.sync_copy(rows0, agg_sp.at[dstbuf.at[j]], add=True)

        @pl.when(j + 2 < CPW)
        def _():
            pltpu.async_copy(g_hbm.at[srcbuf.at[j + 2]], rows0, sem0)

        pltpu.make_async_copy(g_hbm.at[pl.ds(0, CW)], rows1, sem1).wait()
        pltpu.sync_copy(rows1, agg_sp.at[dstbuf.at[j + 1]], add=True)

    plsc.subcore_barrier()

    @pl.loop(0, RPT, step=ROW_BLK)
    def _(b):
        pltpu.sync_copy(agg_sp.at[pl.ds(s * RPT + b, ROW_BLK)], zblk)

        @pl.when(c == 0)
        def _():
            pltpu.sync_copy(zblk, p0_hbm.at[pl.ds(s * RPT + b, ROW_BLK)])

        @pl.when(c == 1)
        def _():
            pltpu.sync_copy(zblk, p1_hbm.at[pl.ds(s * RPT + b, ROW_BLK)])


@jax.jit
def _sc_step(g, src3, dst3):
    mesh = plsc.VectorSubcoreMesh(core_axis_name="c", subcore_axis_name="s")
    f = pl.kernel(
        _sc_step_body,
        out_type=(
            jax.ShapeDtypeStruct((NP, D), jnp.float32),
            jax.ShapeDtypeStruct((NP, D), jnp.float32),
        ),
        compiler_params=pltpu.CompilerParams(use_tc_tiling_on_sc=False),
        mesh=mesh,
        scratch_types=[
            pltpu.VMEM_SHARED((NP, D), jnp.float32),
            pltpu.VMEM((CPW, CW), jnp.int32),
            pltpu.VMEM((CPW, CW), jnp.int32),
            pltpu.VMEM((CW, D), jnp.float32),
            pltpu.VMEM((CW, D), jnp.float32),
            pltpu.VMEM((ROW_BLK, D), jnp.float32),
            pltpu.SemaphoreType.DMA,
            pltpu.SemaphoreType.DMA,
        ],
    )
    return f(g, src3, dst3)


_MLP_BLK = 512


def _tc_mlp_prep_body(x_ref, w1_ref, b1_ref, w2_ref, b2_ref, d0_ref, d1_ref,
                      g0_ref, a1_ref, bb1_ref, a2_ref, bb2_ref):
    h = jnp.dot(x_ref[...], w1_ref[...], preferred_element_type=jnp.float32)
    h = jnp.maximum(h + b1_ref[...], 0.0)
    h = jnp.dot(h, w2_ref[...], preferred_element_type=jnp.float32) + b2_ref[...]
    deg = d0_ref[...] + d1_ref[...]
    norm = lax.rsqrt(jnp.maximum(deg, 1.0))
    g0_ref[...] = h * norm
    a1_ref[...] = (1.0 - ALPHA) * norm * norm
    bb1_ref[...] = (ALPHA * norm) * h
    a2_ref[...] = (1.0 - ALPHA) * norm
    bb2_ref[...] = ALPHA * h


@jax.jit
def _tc_mlp_prep(xp, W1, b1, W2, b2, d0, d1):
    grid = (NP // _MLP_BLK,)
    return pl.pallas_call(
        _tc_mlp_prep_body,
        grid=grid,
        in_specs=[
            pl.BlockSpec((_MLP_BLK, IN_FEATS), lambda i: (i, 0)),
            pl.BlockSpec((IN_FEATS, IN_FEATS), lambda i: (0, 0)),
            pl.BlockSpec((1, IN_FEATS), lambda i: (0, 0)),
            pl.BlockSpec((IN_FEATS, D), lambda i: (0, 0)),
            pl.BlockSpec((1, D), lambda i: (0, 0)),
            pl.BlockSpec((_MLP_BLK, 1), lambda i: (i, 0)),
            pl.BlockSpec((_MLP_BLK, 1), lambda i: (i, 0)),
        ],
        out_specs=(
            pl.BlockSpec((_MLP_BLK, D), lambda i: (i, 0)),
            pl.BlockSpec((_MLP_BLK, 1), lambda i: (i, 0)),
            pl.BlockSpec((_MLP_BLK, D), lambda i: (i, 0)),
            pl.BlockSpec((_MLP_BLK, 1), lambda i: (i, 0)),
            pl.BlockSpec((_MLP_BLK, D), lambda i: (i, 0)),
        ),
        out_shape=(
            jax.ShapeDtypeStruct((NP, D), jnp.float32),
            jax.ShapeDtypeStruct((NP, 1), jnp.float32),
            jax.ShapeDtypeStruct((NP, D), jnp.float32),
            jax.ShapeDtypeStruct((NP, 1), jnp.float32),
            jax.ShapeDtypeStruct((NP, D), jnp.float32),
        ),
    )(xp, W1, b1.reshape(1, IN_FEATS), W2, b2.reshape(1, D), d0, d1)


def _tc_combine_body(p0_ref, p1_ref, a_ref, b_ref, o_ref):
    o_ref[...] = a_ref[...] * (p0_ref[...] + p1_ref[...]) + b_ref[...]


@jax.jit
def _tc_combine(p0, p1, a, b):
    grid = (NP // _MLP_BLK,)
    return pl.pallas_call(
        _tc_combine_body,
        grid=grid,
        in_specs=[
            pl.BlockSpec((_MLP_BLK, D), lambda i: (i, 0)),
            pl.BlockSpec((_MLP_BLK, D), lambda i: (i, 0)),
            pl.BlockSpec((_MLP_BLK, 1), lambda i: (i, 0)),
            pl.BlockSpec((_MLP_BLK, D), lambda i: (i, 0)),
        ],
        out_specs=pl.BlockSpec((_MLP_BLK, D), lambda i: (i, 0)),
        out_shape=jax.ShapeDtypeStruct((NP, D), jnp.float32),
    )(p0, p1, a, b)


def kernel(features, W1, b1, W2, b2, edge_index):
    xp = jnp.pad(features, ((0, NP - N), (0, 0)))
    src = edge_index[0]
    dst = edge_index[1]
    pad_idx = N + (jnp.arange(EP - E, dtype=jnp.int32) % (NP - N))
    src3 = jnp.concatenate([src, pad_idx]).reshape(NW, CPW, CW)
    dst3 = jnp.concatenate([dst, pad_idx]).reshape(NW, CPW, CW)

    d0, d1 = _sc_degree(dst3)
    g, A1, B1, A2, B2 = _tc_mlp_prep(
        xp, W1, b1, W2, b2, d0.reshape(NP, 1), d1.reshape(NP, 1))

    for t in range(K):
        p0, p1 = _sc_step(g, src3, dst3)
        if t < K - 1:
            g = _tc_combine(p0, p1, A1, B1)
        else:
            g = _tc_combine(p0, p1, A2, B2)
    return g[:N]
